# Initial kernel scaffold; baseline (speedup 1.0000x reference)
import jax
import jax.numpy as jnp
from jax.experimental import pallas as pl


def kernel(x, edge_index, hyperedge_index, W1, b1, W2, b2, W3, b3, Wf, bf):
    raise NotImplementedError("write your pallas kernel here")


# speedup vs baseline: 6.8211x; 6.8211x over previous
import functools

import jax
import jax.numpy as jnp
from jax import lax
from jax.experimental import pallas as pl
from jax.experimental.pallas import tpu as pltpu
from jax.experimental.pallas import tpu_sc as plsc

_N = 10000
_NT = 16
_NC = 2
_ROWS = 10112
_RPT = 632
_NCH = 157
_CH = 128
_DUM = 10000
_DH = 64
_P1 = 79


def _sc_mesh():
    return plsc.VectorSubcoreMesh(core_axis_name="c", subcore_axis_name="s")


def _degrees(idx_all):
    @functools.partial(
        pl.kernel,
        out_type=[
            jax.ShapeDtypeStruct((_NC, _ROWS, 16), jnp.float32),
            jax.ShapeDtypeStruct((_ROWS, 16), jnp.float32),
        ],
        mesh=_sc_mesh(),
        compiler_params=pltpu.CompilerParams(use_tc_tiling_on_sc=False),
        scratch_types=[
            pltpu.VMEM_SHARED((_ROWS, 16), jnp.float32),
            pltpu.VMEM_SHARED((_ROWS, 16), jnp.float32),
            pltpu.VMEM((_NCH, _CH), jnp.int32),
            pltpu.VMEM((_NCH, _CH), jnp.int32),
            pltpu.VMEM((_CH, 16), jnp.float32),
            pltpu.VMEM((_RPT, 16), jnp.float32),
        ],
    )
    def deg(idx_hbm, dd_out, invb_out, DT, BT, idxA, idxB, ones_v, vbuf):
        cid = lax.axis_index("c")
        tid = lax.axis_index("s")
        pltpu.sync_copy(idx_hbm.at[cid, 0, tid], idxA)
        pltpu.sync_copy(idx_hbm.at[cid, 1, tid], idxB)

        def fill(r, _):
            ones_v[r, :] = jnp.ones((16,), jnp.float32)
            return 0

        lax.fori_loop(0, _CH, fill, 0)

        def zfill(r, _):
            vbuf[r, :] = jnp.zeros((16,), jnp.float32)
            return 0

        lax.fori_loop(0, _RPT, zfill, 0)
        pltpu.sync_copy(vbuf, DT.at[pl.ds(tid * _RPT, _RPT)])
        pltpu.sync_copy(vbuf, BT.at[pl.ds(tid * _RPT, _RPT)])
        plsc.subcore_barrier()

        def hist_a(c, _):
            pltpu.sync_copy(ones_v, DT.at[idxA.at[c]], add=True)
            return 0

        lax.fori_loop(0, _NCH, hist_a, 0)

        @pl.when(cid == 0)
        def _():
            def hist_d(c, _):
                pltpu.sync_copy(ones_v, DT.at[idxB.at[c]], add=True)
                return 0

            lax.fori_loop(0, _NCH, hist_d, 0)

        @pl.when(cid == 1)
        def _():
            def hist_b(c, _):
                pltpu.sync_copy(ones_v, BT.at[idxB.at[c]], add=True)
                return 0

            lax.fori_loop(0, _NCH, hist_b, 0)

        plsc.subcore_barrier()
        pltpu.sync_copy(DT.at[pl.ds(tid * _RPT, _RPT)],
                        dd_out.at[cid, pl.ds(tid * _RPT, _RPT)])

        @pl.when(cid == 1)
        def _():
            pltpu.sync_copy(BT.at[pl.ds(tid * _RPT, _RPT)], vbuf)

            def inv(r, _):
                vbuf[r, :] = jnp.float32(1.0) / jnp.maximum(
                    vbuf[r, :], jnp.float32(1.0))
                return 0

            lax.fori_loop(0, _RPT, inv, 0)
            pltpu.sync_copy(vbuf, invb_out.at[pl.ds(tid * _RPT, _RPT)])

    return deg(idx_all)


def _conv(xw, idx_all, invb):
    row_chunks = ((0, 128), (128, 128), (256, 128), (384, 128), (512, 120))

    @functools.partial(
        pl.kernel,
        out_type=jax.ShapeDtypeStruct((_NC, 2, _ROWS, _DH), jnp.float32),
        mesh=_sc_mesh(),
        compiler_params=pltpu.CompilerParams(use_tc_tiling_on_sc=False,
                                             needs_layout_passes=False),
        scratch_types=[
            pltpu.VMEM_SHARED((_ROWS, _DH), jnp.float32),
            pltpu.VMEM_SHARED((_ROWS, _DH), jnp.float32),
            pltpu.VMEM((_CH,), jnp.int32),
            pltpu.VMEM((_CH,), jnp.int32),
            pltpu.VMEM((_CH, _DH), jnp.float32),
            pltpu.VMEM((_CH, _DH), jnp.float32),
            pltpu.VMEM((_CH, _DH), jnp.float32),
            pltpu.VMEM((_RPT, 16), jnp.float32),
        ],
    )
    def conv(xw_hbm, idx_hbm, invb_hbm, acc_out,
             T1, T2, idxa, idxb, bufA, bufB, zbuf, invv):
        cid = lax.axis_index("c")
        tid = lax.axis_index("s")
        pltpu.sync_copy(invb_hbm.at[pl.ds(tid * _RPT, _RPT)], invv)

        def zfill(r, _):
            for k in range(4):
                zbuf[r, pl.ds(k * 16, 16)] = jnp.zeros((16,), jnp.float32)
            return 0

        lax.fori_loop(0, _CH, zfill, 0)

        def zero_table(tbl):
            for base, sz in row_chunks:
                pltpu.sync_copy(zbuf.at[pl.ds(0, sz)],
                                tbl.at[pl.ds(tid * _RPT + base, sz)])

        def load_idx(c):
            pltpu.sync_copy(idx_hbm.at[cid, 0, tid, c], idxa)
            pltpu.sync_copy(idx_hbm.at[cid, 1, tid, c], idxb)

        def pair_step(c, _):
            load_idx(c)
            pltpu.sync_copy(T1.at[idxa], bufA)
            pltpu.sync_copy(T1.at[idxb], bufB)

            def avg(r, _):
                for k in range(4):
                    sl = pl.ds(k * 16, 16)
                    bufA[r, sl] = (bufA[r, sl] + bufB[r, sl]) * jnp.float32(0.5)
                return 0

            lax.fori_loop(0, _CH, avg, 0)
            pltpu.sync_copy(bufA, T2.at[idxa], add=True)
            pltpu.sync_copy(bufA, T2.at[idxb], add=True)
            return 0

        for half in range(2):
            zero_table(T2)
            pltpu.sync_copy(xw_hbm.at[half, pl.ds(tid * _RPT, _RPT)],
                            T1.at[pl.ds(tid * _RPT, _RPT)])
            plsc.subcore_barrier()

            @pl.when(cid == 0)
            def _():
                lax.fori_loop(0, _P1, pair_step, 0)

            @pl.when(cid == 1)
            def _():
                def phase_a(c, _):
                    load_idx(c)
                    pltpu.sync_copy(T1.at[idxa], bufA)
                    pltpu.sync_copy(bufA, T2.at[idxb], add=True)
                    return 0

                lax.fori_loop(0, _NCH, phase_a, 0)

            plsc.subcore_barrier()

            @pl.when(cid == 1)
            def _():
                zero_table(T1)
                for base, sz in row_chunks:
                    pltpu.sync_copy(T2.at[pl.ds(tid * _RPT + base, sz)],
                                    bufB.at[pl.ds(0, sz)])

                    def nstep(r, _):
                        s = plsc.load_gather(
                            invv,
                            [jnp.full((16,), base + r, jnp.int32),
                             jnp.zeros((16,), jnp.int32)])
                        for k in range(4):
                            sl = pl.ds(k * 16, 16)
                            bufB[r, sl] = bufB[r, sl] * s
                        return 0

                    lax.fori_loop(0, sz, nstep, 0)
                    pltpu.sync_copy(bufB.at[pl.ds(0, sz)],
                                    T2.at[pl.ds(tid * _RPT + base, sz)])

            plsc.subcore_barrier()

            @pl.when(cid == 0)
            def _():
                lax.fori_loop(_P1, _NCH, pair_step, 0)

            @pl.when(cid == 1)
            def _():
                def phase_b(c, _):
                    load_idx(c)
                    pltpu.sync_copy(T2.at[idxb], bufA)
                    pltpu.sync_copy(bufA, T1.at[idxa], add=True)
                    return 0

                lax.fori_loop(0, _NCH, phase_b, 0)

            plsc.subcore_barrier()

            @pl.when(cid == 0)
            def _():
                pltpu.sync_copy(T2.at[pl.ds(tid * _RPT, _RPT)],
                                acc_out.at[0, half, pl.ds(tid * _RPT, _RPT)])

            @pl.when(cid == 1)
            def _():
                pltpu.sync_copy(T1.at[pl.ds(tid * _RPT, _RPT)],
                                acc_out.at[1, half, pl.ds(tid * _RPT, _RPT)])

            plsc.subcore_barrier()

    return conv(xw, idx_all, invb)


def _mm_in(x, W, b):
    def body(x_ref, w_ref, b_ref, o_ref):
        y = jnp.dot(x_ref[...], w_ref[...],
                    preferred_element_type=jnp.float32) + b_ref[...]
        o_ref[0] = y[:, :_DH]
        o_ref[1] = y[:, _DH:]

    return pl.pallas_call(
        body,
        grid=(10,),
        in_specs=[
            pl.BlockSpec((1000, 128), lambda i: (i, 0)),
            pl.BlockSpec((128, 128), lambda i: (0, 0)),
            pl.BlockSpec((1, 128), lambda i: (0, 0)),
        ],
        out_specs=pl.BlockSpec((2, 1000, _DH), lambda i: (0, i, 0)),
        out_shape=jax.ShapeDtypeStruct((2, _ROWS, _DH), jnp.float32),
    )(x, W, b)


def _relu_h(a_ref, dd_ref):
    iv = 1.0 / jnp.maximum(dd_ref[0][:, :1] + dd_ref[1][:, :1], 1.0)
    h0 = jnp.maximum((a_ref[0, 0] + a_ref[1, 0]) * iv, 0.0)
    h1 = jnp.maximum((a_ref[0, 1] + a_ref[1, 1]) * iv, 0.0)
    return jnp.concatenate([h0, h1], axis=1)


def _mm_mid(acc, dd, W, b):
    def body(a_ref, dd_ref, w_ref, b_ref, o_ref):
        h = _relu_h(a_ref, dd_ref)
        y = jnp.dot(h, w_ref[...],
                    preferred_element_type=jnp.float32) + b_ref[...]
        o_ref[0] = y[:, :_DH]
        o_ref[1] = y[:, _DH:]

    return pl.pallas_call(
        body,
        grid=(10,),
        in_specs=[
            pl.BlockSpec((2, 2, 1000, _DH), lambda i: (0, 0, i, 0)),
            pl.BlockSpec((2, 1000, 16), lambda i: (0, i, 0)),
            pl.BlockSpec((128, 128), lambda i: (0, 0)),
            pl.BlockSpec((1, 128), lambda i: (0, 0)),
        ],
        out_specs=pl.BlockSpec((2, 1000, _DH), lambda i: (0, i, 0)),
        out_shape=jax.ShapeDtypeStruct((2, _ROWS, _DH), jnp.float32),
    )(acc, dd, W, b)


def _mm_head(acc, dd, Wf, bf):
    def body(a_ref, dd_ref, wf_ref, bf_ref, o_ref):
        h = _relu_h(a_ref, dd_ref)
        p = jnp.dot(jnp.sum(h, axis=0, keepdims=True), wf_ref[...],
                    preferred_element_type=jnp.float32) * jnp.float32(1.0 / _N)

        @pl.when(pl.program_id(0) == 0)
        def _():
            o_ref[...] = p + bf_ref[...]

        @pl.when(pl.program_id(0) != 0)
        def _():
            o_ref[...] = o_ref[...] + p

    return pl.pallas_call(
        body,
        grid=(10,),
        in_specs=[
            pl.BlockSpec((2, 2, 1000, _DH), lambda i: (0, 0, i, 0)),
            pl.BlockSpec((2, 1000, 16), lambda i: (0, i, 0)),
            pl.BlockSpec((128, 32), lambda i: (0, 0)),
            pl.BlockSpec((1, 32), lambda i: (0, 0)),
        ],
        out_specs=pl.BlockSpec((1, 32), lambda i: (0, 0)),
        out_shape=jax.ShapeDtypeStruct((1, 32), jnp.float32),
    )(acc, dd, Wf, bf)


def _tile_pad(a):
    pad = jnp.full((_NT * _NCH * _CH - a.shape[0],), _DUM, jnp.int32)
    return jnp.concatenate([a.astype(jnp.int32), pad]).reshape(_NT, _NCH, _CH)


def kernel(x, edge_index, hyperedge_index, W1, b1, W2, b2, W3, b3, Wf, bf):
    idx_all = jnp.stack([
        jnp.stack([_tile_pad(edge_index[0]), _tile_pad(edge_index[1])]),
        jnp.stack([_tile_pad(hyperedge_index[0]),
                   _tile_pad(hyperedge_index[1])]),
    ])
    dd, invb = _degrees(idx_all)
    h = _mm_in(x, W1, b1.reshape(1, -1))
    h = _conv(h, idx_all, invb)
    h = _mm_mid(h, dd, W2, b2.reshape(1, -1))
    h = _conv(h, idx_all, invb)
    h = _mm_mid(h, dd, W3, b3.reshape(1, -1))
    h = _conv(h, idx_all, invb)
    out = _mm_head(h, dd, Wf, bf.reshape(1, -1))
    return out.reshape(32)
